# baseline (speedup 1.0000x reference)
import jax
import jax.numpy as jnp
from jax.experimental import pallas as pl
from jax.experimental.pallas import tpu as pltpu

_B, _C, _W, _H = 32, 19, 128, 128
_WB = 16
_NBLK = _W // _WB


def _body(x_ref, o_ref):
    data = x_ref[0]
    s = jnp.sum(data, axis=0, keepdims=True)
    nd = data / s
    o_ref[0] = jnp.transpose(nd, (2, 1, 0))


def kernel(x):
    out4 = pl.pallas_call(
        _body,
        grid=(_B, _NBLK),
        in_specs=[pl.BlockSpec((1, _C, _WB, _H), lambda b, j: (b, 0, j, 0))],
        out_specs=pl.BlockSpec((1, _H, _WB, _C), lambda b, j: (b, 0, j, 0)),
        out_shape=jax.ShapeDtypeStruct((_B, _H, _W, _C), jnp.float32),
        compiler_params=pltpu.CompilerParams(
            dimension_semantics=("arbitrary", "arbitrary"),
        ),
    )(x)
    return out4.reshape(_B * _W * _H, _C)
